# Initial kernel scaffold; baseline (speedup 1.0000x reference)
import jax
import jax.numpy as jnp
from jax.experimental import pallas as pl


def kernel(x, edge_index, edge_weight, energy, W, b):
    raise NotImplementedError("write your pallas kernel here")


# speedup vs baseline: 8.8708x; 8.8708x over previous
import functools

import jax
import jax.numpy as jnp
from jax import lax
from jax.experimental import pallas as pl
from jax.experimental.pallas import tpu as pltpu
from jax.experimental.pallas import tpu_sc as plsc

N = 10000
E = 320000
D = 128

NC = 2
NS = 16
NW = NC * NS
C = 128
CHUNKS = 80
EPW = C * CHUNKS
E_PAD = EPW * NW
NP = 10240
ROWS_PER_TILE = NP // NS


def _mm_body(x_ref, wt_ref, b_ref, o_ref):
    o_ref[...] = (
        jnp.dot(x_ref[...], wt_ref[...], preferred_element_type=jnp.float32)
        + b_ref[...]
    )


def _linear(x, W, b):
    blk = 1000
    return pl.pallas_call(
        _mm_body,
        grid=(N // blk,),
        in_specs=[
            pl.BlockSpec((blk, D), lambda i: (i, 0)),
            pl.BlockSpec((D, D), lambda i: (0, 0)),
            pl.BlockSpec((1, D), lambda i: (0, 0)),
        ],
        out_specs=pl.BlockSpec((blk, D), lambda i: (i, 0)),
        out_shape=jax.ShapeDtypeStruct((N, D), jnp.float32),
    )(x, W.T, b.reshape(1, D))


def _sc_body(h_hbm, src_hbm, dst_hbm, ew_hbm, en_hbm, zero_hbm, pout_hbm,
             acc, en_v, srcv, dstv, eww, wv, rows_v, sem):
    cid = lax.axis_index("c")
    sid = lax.axis_index("s")
    wid = sid * NC + cid
    base = wid * EPW

    pltpu.sync_copy(en_hbm, en_v)
    rows_off = pl.multiple_of(sid * ROWS_PER_TILE, 8)
    pltpu.sync_copy(
        zero_hbm.at[pl.ds(rows_off, ROWS_PER_TILE)],
        acc.at[pl.ds(rows_off, ROWS_PER_TILE)],
    )
    plsc.subcore_barrier()

    def chunk_body(c, carry):
        off = base + c * C
        pltpu.sync_copy(src_hbm.at[pl.ds(off, C)], srcv)
        pltpu.sync_copy(dst_hbm.at[pl.ds(off, C)], dstv)
        pltpu.sync_copy(ew_hbm.at[pl.ds(off, C)], eww)
        gather = pltpu.async_copy(h_hbm.at[srcv], rows_v, sem)
        for i in range(C // 16):
            sl = pl.ds(i * 16, 16)
            es = plsc.load_gather(en_v, [srcv[sl]])
            ed = plsc.load_gather(en_v, [dstv[sl]])
            ee = jnp.clip(es, 0.0, 10.0) + jnp.clip(ed, 0.0, 10.0)
            wv[sl] = eww[sl] / (1.0 + jnp.exp(ee))
        gather.wait()

        def scale_group(g, _):
            wg = wv[pl.ds(g * 16, 16)]
            for i in range(16):
                wb = jnp.full((16,), wg[i], dtype=jnp.float32)
                row = g * 16 + i
                for j in range(D // 16):
                    sl = pl.ds(j * 16, 16)
                    rows_v[row, sl] = rows_v[row, sl] * wb
            return _

        lax.fori_loop(0, C // 16, scale_group, 0)
        pltpu.sync_copy(rows_v, acc.at[dstv], add=True)
        return carry

    lax.fori_loop(0, CHUNKS, chunk_body, 0)
    plsc.subcore_barrier()
    pltpu.sync_copy(
        acc.at[pl.ds(rows_off, ROWS_PER_TILE)],
        pout_hbm.at[cid, pl.ds(rows_off, ROWS_PER_TILE)],
    )


def _sc_aggregate(h, src, dst, ew, en, zero):
    mesh = plsc.VectorSubcoreMesh(core_axis_name="c", subcore_axis_name="s")
    k = pl.kernel(
        _sc_body,
        out_type=jax.ShapeDtypeStruct((NC, NP, D), jnp.float32),
        mesh=mesh,
        scratch_types=[
            pltpu.VMEM_SHARED((NP, D), jnp.float32),
            pltpu.VMEM((N,), jnp.float32),
            pltpu.VMEM((C,), jnp.int32),
            pltpu.VMEM((C,), jnp.int32),
            pltpu.VMEM((C,), jnp.float32),
            pltpu.VMEM((C,), jnp.float32),
            pltpu.VMEM((C, D), jnp.float32),
            pltpu.SemaphoreType.DMA,
        ],
        compiler_params=pltpu.CompilerParams(needs_layout_passes=False),
    )
    return k(h, src, dst, ew, en, zero)


def _add_body(p_ref, o_ref):
    o_ref[...] = p_ref[0] + p_ref[1]


def _sum_partials(p):
    blk = 1000
    return pl.pallas_call(
        _add_body,
        grid=(N // blk,),
        in_specs=[pl.BlockSpec((NC, blk, D), lambda i: (0, i, 0))],
        out_specs=pl.BlockSpec((blk, D), lambda i: (i, 0)),
        out_shape=jax.ShapeDtypeStruct((N, D), jnp.float32),
    )(p)


@jax.jit
def kernel(x, edge_index, edge_weight, energy, W, b):
    h = _linear(x, W, b)
    pad = E_PAD - E
    src = jnp.concatenate([edge_index[0].astype(jnp.int32),
                           jnp.zeros((pad,), jnp.int32)])
    dst = jnp.concatenate([edge_index[1].astype(jnp.int32),
                           jnp.zeros((pad,), jnp.int32)])
    ew = jnp.concatenate([edge_weight, jnp.zeros((pad,), jnp.float32)])
    en = energy.reshape(N)
    zero = jnp.zeros((NP, D), jnp.float32)
    partials = _sc_aggregate(h, src, dst, ew, en, zero)
    return _sum_partials(partials)


# speedup vs baseline: 11.2395x; 1.2670x over previous
import functools

import jax
import jax.numpy as jnp
from jax import lax
from jax.experimental import pallas as pl
from jax.experimental.pallas import tpu as pltpu
from jax.experimental.pallas import tpu_sc as plsc

N = 10000
E = 320000
D = 128

NC = 2
NS = 16
NW = NC * NS
C = 128
CHUNKS = 80
EPW = C * CHUNKS
E_PAD = EPW * NW
NP = 10240
ROWS_PER_TILE = NP // NS


def _mm_body(x_ref, wt_ref, b_ref, o_ref):
    o_ref[...] = (
        jnp.dot(x_ref[...], wt_ref[...], preferred_element_type=jnp.float32)
        + b_ref[...]
    )


def _linear(x, W, b):
    blk = 1000
    return pl.pallas_call(
        _mm_body,
        grid=(N // blk,),
        in_specs=[
            pl.BlockSpec((blk, D), lambda i: (i, 0)),
            pl.BlockSpec((D, D), lambda i: (0, 0)),
            pl.BlockSpec((1, D), lambda i: (0, 0)),
        ],
        out_specs=pl.BlockSpec((blk, D), lambda i: (i, 0)),
        out_shape=jax.ShapeDtypeStruct((N, D), jnp.float32),
    )(x, W.T, b.reshape(1, D))


PAIRS = CHUNKS // 2


def _sc_body(h_hbm, src_hbm, dst_hbm, ew_hbm, en_hbm, zero_hbm, pout_hbm,
             acc, en_v, srcv, dstv, dsts, eww, wv, rows_v,
             sem_i0, sem_i1, sem_g0, sem_g1, sem_s0, sem_s1):
    cid = lax.axis_index("c")
    sid = lax.axis_index("s")
    wid = sid * NC + cid
    base = wid * EPW
    sem_i = (sem_i0, sem_i1)
    sem_g = (sem_g0, sem_g1)
    sem_s = (sem_s0, sem_s1)

    pltpu.sync_copy(en_hbm, en_v)
    rows_off = pl.multiple_of(sid * ROWS_PER_TILE, 8)
    pltpu.sync_copy(
        zero_hbm.at[pl.ds(rows_off, ROWS_PER_TILE)],
        acc.at[pl.ds(rows_off, ROWS_PER_TILE)],
    )
    plsc.subcore_barrier()

    def fire_idx(k, c):
        off = base + c * C
        pltpu.async_copy(src_hbm.at[pl.ds(off, C)], srcv.at[k], sem_i[k])
        pltpu.async_copy(dst_hbm.at[pl.ds(off, C)], dstv.at[k], sem_i[k])
        pltpu.async_copy(ew_hbm.at[pl.ds(off, C)], eww.at[k], sem_i[k])

    def wait_idx(k):
        pltpu.make_async_copy(src_hbm.at[pl.ds(0, C)], srcv.at[k], sem_i[k]).wait()
        pltpu.make_async_copy(dst_hbm.at[pl.ds(0, C)], dstv.at[k], sem_i[k]).wait()
        pltpu.make_async_copy(ew_hbm.at[pl.ds(0, C)], eww.at[k], sem_i[k]).wait()

    def fire_gather(k):
        pltpu.async_copy(h_hbm.at[srcv.at[k]], rows_v.at[k], sem_g[k])

    def wait_gather(k):
        pltpu.make_async_copy(h_hbm.at[srcv.at[k]], rows_v.at[k], sem_g[k]).wait()

    def fire_scatter(k):
        pltpu.async_copy(rows_v.at[k], acc.at[dsts.at[k]], sem_s[k], add=True)

    def wait_scatter(k):
        pltpu.make_async_copy(rows_v.at[k], acc.at[dsts.at[k]], sem_s[k]).wait()

    def half(k, p):
        c = 2 * p + k
        for i in range(C // 16):
            sl = pl.ds(i * 16, 16)
            es = plsc.load_gather(en_v, [srcv[k, sl]])
            ed = plsc.load_gather(en_v, [dstv[k, sl]])
            ee = jnp.clip(es, 0.0, 10.0) + jnp.clip(ed, 0.0, 10.0)
            wv[k, sl] = eww[k, sl] / (1.0 + jnp.exp(ee))
        wait_gather(k)

        def scale_group(g, _):
            wg = wv[k, pl.ds(g * 16, 16)]
            for i in range(16):
                wb = jnp.full((16,), wg[i], dtype=jnp.float32)
                row = g * 16 + i
                for j in range(D // 16):
                    sl = pl.ds(j * 16, 16)
                    rows_v[k, row, sl] = rows_v[k, row, sl] * wb
            return _

        lax.fori_loop(0, C // 16, scale_group, 0)
        for i in range(C // 16):
            sl = pl.ds(i * 16, 16)
            dsts[k, sl] = dstv[k, sl]
        if k == 0:
            @pl.when(p < PAIRS - 1)
            def _():
                fire_idx(0, c + 2)

            wait_idx(1)

            @pl.when(p > 0)
            def _():
                wait_scatter(1)

            fire_gather(1)
        else:
            @pl.when(p < PAIRS - 1)
            def _():
                fire_idx(1, c + 2)
                wait_idx(0)
                wait_scatter(0)
                fire_gather(0)

        fire_scatter(k)

    def pair_body(p, carry):
        half(0, p)
        half(1, p)
        return carry

    fire_idx(0, 0)
    wait_idx(0)
    fire_gather(0)
    fire_idx(1, 1)
    lax.fori_loop(0, PAIRS, pair_body, 0)
    wait_scatter(0)
    wait_scatter(1)
    plsc.subcore_barrier()
    pltpu.sync_copy(
        acc.at[pl.ds(rows_off, ROWS_PER_TILE)],
        pout_hbm.at[cid, pl.ds(rows_off, ROWS_PER_TILE)],
    )


def _sc_aggregate(h, src, dst, ew, en, zero):
    mesh = plsc.VectorSubcoreMesh(core_axis_name="c", subcore_axis_name="s")
    k = pl.kernel(
        _sc_body,
        out_type=jax.ShapeDtypeStruct((NC, NP, D), jnp.float32),
        mesh=mesh,
        scratch_types=[
            pltpu.VMEM_SHARED((NP, D), jnp.float32),
            pltpu.VMEM((N,), jnp.float32),
            pltpu.VMEM((2, C), jnp.int32),
            pltpu.VMEM((2, C), jnp.int32),
            pltpu.VMEM((2, C), jnp.int32),
            pltpu.VMEM((2, C), jnp.float32),
            pltpu.VMEM((2, C), jnp.float32),
            pltpu.VMEM((2, C, D), jnp.float32),
            pltpu.SemaphoreType.DMA,
            pltpu.SemaphoreType.DMA,
            pltpu.SemaphoreType.DMA,
            pltpu.SemaphoreType.DMA,
            pltpu.SemaphoreType.DMA,
            pltpu.SemaphoreType.DMA,
        ],
        compiler_params=pltpu.CompilerParams(needs_layout_passes=False),
    )
    return k(h, src, dst, ew, en, zero)


def _add_body(p_ref, o_ref):
    o_ref[...] = p_ref[0] + p_ref[1]


def _sum_partials(p):
    blk = 1000
    return pl.pallas_call(
        _add_body,
        grid=(N // blk,),
        in_specs=[pl.BlockSpec((NC, blk, D), lambda i: (0, i, 0))],
        out_specs=pl.BlockSpec((blk, D), lambda i: (i, 0)),
        out_shape=jax.ShapeDtypeStruct((N, D), jnp.float32),
    )(p)


@jax.jit
def kernel(x, edge_index, edge_weight, energy, W, b):
    h = _linear(x, W, b)
    pad = E_PAD - E
    src = jnp.concatenate([edge_index[0].astype(jnp.int32),
                           jnp.zeros((pad,), jnp.int32)])
    dst = jnp.concatenate([edge_index[1].astype(jnp.int32),
                           jnp.zeros((pad,), jnp.int32)])
    ew = jnp.concatenate([edge_weight, jnp.zeros((pad,), jnp.float32)])
    en = energy.reshape(N)
    zero = jnp.zeros((NP, D), jnp.float32)
    partials = _sc_aggregate(h, src, dst, ew, en, zero)
    return _sum_partials(partials)


# speedup vs baseline: 12.2606x; 1.0909x over previous
import functools

import jax
import jax.numpy as jnp
from jax import lax
from jax.experimental import pallas as pl
from jax.experimental.pallas import tpu as pltpu
from jax.experimental.pallas import tpu_sc as plsc

N = 10000
E = 320000
D = 128

NC = 2
NS = 16
NW = NC * NS
C = 128
PAIRS0 = 58
PAIRS1 = 22
PAIRS_TOTAL = PAIRS0 + PAIRS1
E_PAD = NS * 2 * C * PAIRS_TOTAL
NP = 10240
ROWS_PER_TILE = NP // NS


def _mm_body(x_ref, wt_ref, b_ref, o_ref):
    o_ref[...] = (
        jnp.dot(x_ref[...], wt_ref[...], preferred_element_type=jnp.float32)
        + b_ref[...]
    )


def _linear(x, W, b):
    blk = 1000
    return pl.pallas_call(
        _mm_body,
        grid=(N // blk,),
        in_specs=[
            pl.BlockSpec((blk, D), lambda i: (i, 0)),
            pl.BlockSpec((D, D), lambda i: (0, 0)),
            pl.BlockSpec((1, D), lambda i: (0, 0)),
        ],
        out_specs=pl.BlockSpec((blk, D), lambda i: (i, 0)),
        out_shape=jax.ShapeDtypeStruct((N, D), jnp.float32),
    )(x, W.T, b.reshape(1, D))


def _sc_body(h_hbm, src_hbm, dst_hbm, ew_hbm, en_hbm, zero_hbm, pout_hbm,
             acc, en_v, srcv, dstv, dsts, eww, wv, rows_v,
             sem_i0, sem_i1, sem_g0, sem_g1, sem_s0, sem_s1):
    cid = lax.axis_index("c")
    sid = lax.axis_index("s")
    core0_total = NS * 2 * C * PAIRS0
    pairs = jnp.where(cid == 0, PAIRS0, PAIRS1)
    base = jnp.where(cid == 0, sid * (2 * C * PAIRS0),
                     core0_total + sid * (2 * C * PAIRS1))
    sem_i = (sem_i0, sem_i1)
    sem_g = (sem_g0, sem_g1)
    sem_s = (sem_s0, sem_s1)

    pltpu.sync_copy(en_hbm, en_v)
    rows_off = pl.multiple_of(sid * ROWS_PER_TILE, 8)
    pltpu.sync_copy(
        zero_hbm.at[pl.ds(rows_off, ROWS_PER_TILE)],
        acc.at[pl.ds(rows_off, ROWS_PER_TILE)],
    )
    plsc.subcore_barrier()

    def fire_idx(k, c):
        off = base + c * C
        pltpu.async_copy(src_hbm.at[pl.ds(off, C)], srcv.at[k], sem_i[k])
        pltpu.async_copy(dst_hbm.at[pl.ds(off, C)], dstv.at[k], sem_i[k])
        pltpu.async_copy(ew_hbm.at[pl.ds(off, C)], eww.at[k], sem_i[k])

    def wait_idx(k):
        pltpu.make_async_copy(src_hbm.at[pl.ds(0, C)], srcv.at[k], sem_i[k]).wait()
        pltpu.make_async_copy(dst_hbm.at[pl.ds(0, C)], dstv.at[k], sem_i[k]).wait()
        pltpu.make_async_copy(ew_hbm.at[pl.ds(0, C)], eww.at[k], sem_i[k]).wait()

    def fire_gather(k):
        pltpu.async_copy(h_hbm.at[srcv.at[k]], rows_v.at[k], sem_g[k])

    def wait_gather(k):
        pltpu.make_async_copy(h_hbm.at[srcv.at[k]], rows_v.at[k], sem_g[k]).wait()

    def fire_scatter(k):
        pltpu.async_copy(rows_v.at[k], acc.at[dsts.at[k]], sem_s[k], add=True)

    def wait_scatter(k):
        pltpu.make_async_copy(rows_v.at[k], acc.at[dsts.at[k]], sem_s[k]).wait()

    def half(k, p):
        c = 2 * p + k
        for i in range(C // 16):
            sl = pl.ds(i * 16, 16)
            es = plsc.load_gather(en_v, [srcv[k, sl]])
            ed = plsc.load_gather(en_v, [dstv[k, sl]])
            ee = jnp.clip(es, 0.0, 10.0) + jnp.clip(ed, 0.0, 10.0)
            wv[k, sl] = eww[k, sl] / (1.0 + jnp.exp(ee))
        wait_gather(k)

        def scale_group(g, _):
            wg = wv[k, pl.ds(g * 16, 16)]
            for i in range(16):
                wb = jnp.full((16,), wg[i], dtype=jnp.float32)
                row = g * 16 + i
                for j in range(D // 16):
                    sl = pl.ds(j * 16, 16)
                    rows_v[k, row, sl] = rows_v[k, row, sl] * wb
            return _

        lax.fori_loop(0, C // 16, scale_group, 0)
        for i in range(C // 16):
            sl = pl.ds(i * 16, 16)
            dsts[k, sl] = dstv[k, sl]
        if k == 0:
            @pl.when(p < pairs - 1)
            def _():
                fire_idx(0, c + 2)

            wait_idx(1)

            @pl.when(p > 0)
            def _():
                wait_scatter(1)

            fire_gather(1)
        else:
            @pl.when(p < pairs - 1)
            def _():
                fire_idx(1, c + 2)
                wait_idx(0)
                wait_scatter(0)
                fire_gather(0)

        fire_scatter(k)

    def pair_body(p, carry):
        half(0, p)
        half(1, p)
        return carry

    fire_idx(0, 0)
    wait_idx(0)
    fire_gather(0)
    fire_idx(1, 1)
    lax.fori_loop(0, pairs, pair_body, 0)
    wait_scatter(0)
    wait_scatter(1)
    plsc.subcore_barrier()
    pltpu.sync_copy(
        acc.at[pl.ds(rows_off, ROWS_PER_TILE)],
        pout_hbm.at[cid, pl.ds(rows_off, ROWS_PER_TILE)],
    )


def _sc_aggregate(h, src, dst, ew, en, zero):
    mesh = plsc.VectorSubcoreMesh(core_axis_name="c", subcore_axis_name="s")
    k = pl.kernel(
        _sc_body,
        out_type=jax.ShapeDtypeStruct((NC, NP, D), jnp.float32),
        mesh=mesh,
        scratch_types=[
            pltpu.VMEM_SHARED((NP, D), jnp.float32),
            pltpu.VMEM((N,), jnp.float32),
            pltpu.VMEM((2, C), jnp.int32),
            pltpu.VMEM((2, C), jnp.int32),
            pltpu.VMEM((2, C), jnp.int32),
            pltpu.VMEM((2, C), jnp.float32),
            pltpu.VMEM((2, C), jnp.float32),
            pltpu.VMEM((2, C, D), jnp.float32),
            pltpu.SemaphoreType.DMA,
            pltpu.SemaphoreType.DMA,
            pltpu.SemaphoreType.DMA,
            pltpu.SemaphoreType.DMA,
            pltpu.SemaphoreType.DMA,
            pltpu.SemaphoreType.DMA,
        ],
        compiler_params=pltpu.CompilerParams(needs_layout_passes=False),
    )
    return k(h, src, dst, ew, en, zero)


def _add_body(p_ref, o_ref):
    o_ref[...] = p_ref[0] + p_ref[1]


def _sum_partials(p):
    blk = 1000
    return pl.pallas_call(
        _add_body,
        grid=(N // blk,),
        in_specs=[pl.BlockSpec((NC, blk, D), lambda i: (0, i, 0))],
        out_specs=pl.BlockSpec((blk, D), lambda i: (i, 0)),
        out_shape=jax.ShapeDtypeStruct((N, D), jnp.float32),
    )(p)


@jax.jit
def kernel(x, edge_index, edge_weight, energy, W, b):
    h = _linear(x, W, b)
    pad = E_PAD - E
    src = jnp.concatenate([edge_index[0].astype(jnp.int32),
                           jnp.zeros((pad,), jnp.int32)])
    dst = jnp.concatenate([edge_index[1].astype(jnp.int32),
                           jnp.zeros((pad,), jnp.int32)])
    ew = jnp.concatenate([edge_weight, jnp.zeros((pad,), jnp.float32)])
    en = energy.reshape(N)
    zero = jnp.zeros((NP, D), jnp.float32)
    partials = _sc_aggregate(h, src, dst, ew, en, zero)
    return _sum_partials(partials)
